# baseline (speedup 1.0000x reference)
import functools

import numpy as np
import jax
import jax.numpy as jnp
from jax import lax
from jax.experimental import pallas as pl
from jax.experimental.pallas import tpu as pltpu
from jax.experimental.pallas import tpu_sc as plsc

_D = 288
_NQ = 256
_NS = 16
_B = 8
_N = 2048
_EPS = 1e-5
_R = 0.3
_TN = 1024


def _bf16_dot(a, w_ref):
    return jnp.dot(a.astype(jnp.bfloat16), w_ref[...],
                   preferred_element_type=jnp.float32)


def _k1_body(x_ref, ex_ref, w1_ref, b1_ref, g1_ref, be1_ref,
             w2_ref, b2_ref, g2_ref, be2_ref, w3a_ref, b3a_ref,
             w3d_ref, b3d_ref, at_ref, sag1_ref, vote_ref, hpre_ref):
    x = x_ref[0]
    h = (_bf16_dot(x, w1_ref) + b1_ref[...]) * g1_ref[...] + be1_ref[...]
    h = jnp.maximum(h, 0.0)
    h = (_bf16_dot(h, w2_ref) + b2_ref[...]) * g2_ref[...] + be2_ref[...]
    h = jnp.maximum(h, 0.0)
    out3 = _bf16_dot(h, w3a_ref) + b3a_ref[...]
    outd = _bf16_dot(h, w3d_ref) + b3d_ref[...]
    raw = x + outd
    ssq = jnp.sum(raw * raw, axis=1, keepdims=True)
    feats = raw / jnp.sqrt(ssq)
    hpre_ref[...] = _bf16_dot(feats, at_ref) * sag1_ref[...]
    shift = jnp.clip(jax.nn.sigmoid(out3), 0.1, 0.9)
    vote_ref[0] = ex_ref[0] + shift - 0.5


def _fps_body(xs_ref, ys_ref, zs_ref, out_ref):
    xs = xs_ref[...]
    ys = ys_ref[...]
    zs = zs_ref[...]
    iota = lax.broadcasted_iota(jnp.int32, (_B, _N), 1)
    iota_q = lax.broadcasted_iota(jnp.int32, (_B, _NQ), 1)

    def step(i, carry):
        dists, far = carry
        out_ref[...] = jnp.where(iota_q == i,
                                 jnp.broadcast_to(far, (_B, _NQ)),
                                 out_ref[...])
        oh = iota == far
        cx = jnp.sum(jnp.where(oh, xs, 0.0), axis=1, keepdims=True)
        cy = jnp.sum(jnp.where(oh, ys, 0.0), axis=1, keepdims=True)
        cz = jnp.sum(jnp.where(oh, zs, 0.0), axis=1, keepdims=True)
        d = (xs - cx) ** 2 + (ys - cy) ** 2 + (zs - cz) ** 2
        dists = jnp.minimum(dists, d)
        mx = jnp.max(dists, axis=1, keepdims=True)
        far = jnp.min(jnp.where(dists == mx, iota, _N), axis=1, keepdims=True)
        return dists, far

    lax.fori_loop(0, _NQ, step,
                  (jnp.full((_B, _N), 1e10, jnp.float32),
                   jnp.zeros((_B, 1), jnp.int32)))


def _ballq_body(nb, vx_hbm, vy_hbm, vz_hbm, sinds_hbm, hrows_hbm,
                nxyz_hbm, gx_hbm, gf_hbm,
                px, py, pz, sq, qx, qy, qz, nbuf, lst, gxbuf, glist,
                rb0, rb1, rb2, rb3, gsems, osems):
    wpb = 32 // nb
    _QW = _NQ // wpb
    wid = lax.axis_index("s") * 2 + lax.axis_index("c")
    b = wid // wpb
    s0 = (wid % wpb) * _QW
    pltpu.sync_copy(vx_hbm.at[b], px)
    pltpu.sync_copy(vy_hbm.at[b], py)
    pltpu.sync_copy(vz_hbm.at[b], pz)
    pltpu.sync_copy(sinds_hbm.at[b, pl.ds(s0, _QW)], sq)

    lanes = lax.broadcasted_iota(jnp.int32, (16,), 0)
    zeros16 = jnp.zeros((16,), jnp.int32)
    ones16 = jnp.full((16,), 1, jnp.int32)
    twos16 = jnp.full((16,), 2, jnp.int32)

    for g in range(_QW // 16):
        qi = sq[pl.ds(g * 16, 16)]
        gqx = plsc.load_gather(px, [qi])
        gqy = plsc.load_gather(py, [qi])
        gqz = plsc.load_gather(pz, [qi])
        qx[pl.ds(g * 16, 16)] = gqx
        qy[pl.ds(g * 16, 16)] = gqy
        qz[pl.ds(g * 16, 16)] = gqz
        rid = lanes + g * 16
        plsc.store_scatter(nbuf, [rid, zeros16], gqx)
        plsc.store_scatter(nbuf, [rid, ones16], gqy)
        plsc.store_scatter(nbuf, [rid, twos16], gqz)
    pltpu.sync_copy(nbuf, nxyz_hbm.at[b, pl.ds(s0, _QW), :])

    r2 = jnp.float32(_R * _R)
    inv_r = jnp.float32(_R)
    _UNROLL = 8

    def per_query(qq, _):
        qxb = jnp.full((16,), qx[pl.ds(qq, 16)][0], jnp.float32)
        qyb = jnp.full((16,), qy[pl.ds(qq, 16)][0], jnp.float32)
        qzb = jnp.full((16,), qz[pl.ds(qq, 16)][0], jnp.float32)

        def chunk(cu, cursor):
            for u in range(_UNROLL):
                off = cu * (16 * _UNROLL) + u * 16
                pxv = px[pl.ds(off, 16)]
                pyv = py[pl.ds(off, 16)]
                pzv = pz[pl.ds(off, 16)]
                dx = pxv - qxb
                dy = pyv - qyb
                dz = pzv - qzb
                d2 = dx * dx + dy * dy + dz * dz
                m = d2 < r2
                cnt = plsc.all_reduce_population_count(m)[0]
                slot = jnp.minimum(cursor, 16)
                plsc.store_compressed(lst.at[pl.ds(slot, 16)],
                                      lanes + off, mask=m)
                cursor = cursor + cnt
            return cursor

        cursor = lax.fori_loop(0, _N // (16 * _UNROLL), chunk, jnp.int32(0))
        cnt16 = jnp.minimum(cursor, 16)
        nbv = lst[pl.ds(0, 16)]
        first = jnp.full((16,), nbv[0], jnp.int32)
        nb = jnp.where(lanes < cnt16, nbv, first)
        gxx = (plsc.load_gather(px, [nb]) - qxb) / inv_r
        gxy = (plsc.load_gather(py, [nb]) - qyb) / inv_r
        gxz = (plsc.load_gather(pz, [nb]) - qzb) / inv_r
        qqv = jnp.full((16,), qq, jnp.int32)
        plsc.store_scatter(gxbuf, [lanes, qqv, zeros16], gxx)
        plsc.store_scatter(gxbuf, [lanes, qqv, ones16], gxy)
        plsc.store_scatter(gxbuf, [lanes, qqv, twos16], gxz)
        plsc.store_scatter(glist, [lanes * _QW + qqv], nb + b * _N)
        return 0

    lax.fori_loop(0, _QW, per_query, 0)
    pltpu.sync_copy(gxbuf, gx_hbm.at[b, :, pl.ds(s0, _QW), :])

    bufs = [rb0, rb1, rb2, rb3]

    def _gather(k):
        return pltpu.async_copy(
            hrows_hbm.at[glist.at[pl.ds(k * _QW, _QW)]],
            bufs[k % 4], gsems.at[k % 4])

    def _out(k):
        return pltpu.async_copy(
            bufs[k % 4], gf_hbm.at[b, k, pl.ds(s0, _QW), :],
            osems.at[k % 4])

    gcp = [None] * _NS
    ocp = [None] * _NS
    for k in range(4):
        gcp[k] = _gather(k)
    for k in range(_NS):
        gcp[k].wait()
        ocp[k] = _out(k)
        nk = k + 2
        if 4 <= nk < _NS:
            ocp[nk - 4].wait()
            gcp[nk] = _gather(nk)
    for k in range(_NS - 4, _NS):
        ocp[k].wait()


def _k4_body(gf_ref, gx_ref, b3t_ref, sag1_ref, sab1_ref, w2_ref, sag2_ref,
             sab2_ref, w3_ref, sag3_ref, sab3_ref, out_ref):
    acc = jnp.zeros((_NQ, _D), jnp.float32)
    b3t = b3t_ref[...]
    for k in range(_NS):
        g = gf_ref[0, k]
        gxk = gx_ref[0, k].astype(jnp.bfloat16).astype(jnp.float32)
        xyzp = (gxk[:, 0:1] * b3t[0:1, :] + gxk[:, 1:2] * b3t[1:2, :]
                + gxk[:, 2:3] * b3t[2:3, :]) * sag1_ref[...]
        l1 = jnp.maximum(g + xyzp + sab1_ref[...], 0.0)
        l2 = jnp.maximum(
            _bf16_dot(l1, w2_ref) * sag2_ref[...] + sab2_ref[...], 0.0)
        l3 = jnp.maximum(
            _bf16_dot(l2, w3_ref) * sag3_ref[...] + sab3_ref[...], 0.0)
        acc = jnp.maximum(acc, l3)
    out_ref[0] = acc


def _make_ballq(nb):
    qw = _NQ // (32 // nb)
    mesh = plsc.VectorSubcoreMesh(core_axis_name="c", subcore_axis_name="s")
    return functools.partial(
        pl.kernel,
        out_type=(
            jax.ShapeDtypeStruct((nb, _NQ, 3), jnp.float32),
            jax.ShapeDtypeStruct((nb, _NS, _NQ, 3), jnp.float32),
            jax.ShapeDtypeStruct((nb, _NS, _NQ, _D), jnp.float32),
        ),
        mesh=mesh,
        compiler_params=pltpu.CompilerParams(needs_layout_passes=False,
                                             use_tc_tiling_on_sc=False),
        scratch_types=[
            pltpu.VMEM((_N,), jnp.float32),
            pltpu.VMEM((_N,), jnp.float32),
            pltpu.VMEM((_N,), jnp.float32),
            pltpu.VMEM((qw,), jnp.int32),
            pltpu.VMEM((qw + 16,), jnp.float32),
            pltpu.VMEM((qw + 16,), jnp.float32),
            pltpu.VMEM((qw + 16,), jnp.float32),
            pltpu.VMEM((qw, 3), jnp.float32),
            pltpu.VMEM((32,), jnp.int32),
            pltpu.VMEM((_NS, qw, 3), jnp.float32),
            pltpu.VMEM((_NS * qw,), jnp.int32),
            pltpu.VMEM((qw, _D), jnp.float32),
            pltpu.VMEM((qw, _D), jnp.float32),
            pltpu.VMEM((qw, _D), jnp.float32),
            pltpu.VMEM((qw, _D), jnp.float32),
            pltpu.SemaphoreType.DMA((4,)),
            pltpu.SemaphoreType.DMA((4,)),
        ],
    )(functools.partial(_ballq_body, nb))


def kernel(encode_xyz, encode_features, W1, b1, g1, be1, W2, b2, g2, be2,
           W3, b3, saW1, sag1, sab1, saW2, sag2, sab2, saW3, sag3, sab3):
    f32 = jnp.float32
    bf16 = jnp.bfloat16
    scale = f32(1.0 / np.sqrt(1.0 + _EPS))
    g1s = (g1 * scale)[None, :]
    g2s = (g2 * scale)[None, :]
    sag1s = (sag1 * scale)[None, :]
    sag2s = (sag2 * scale)[None, :]
    sag3s = (sag3 * scale)[None, :]

    xT = jnp.transpose(encode_features, (0, 2, 1))
    w1t = W1.T.astype(bf16)
    w2t = W2.T.astype(bf16)
    w3a = W3[:3].T.astype(bf16)
    b3a = b3[:3][None, :]
    w3d = W3[3:].T.astype(bf16)
    b3d = b3[3:][None, :]
    at = saW1[:, 3:].T.astype(bf16)
    b3t = saW1[:, :3].T.astype(bf16).astype(f32)
    saw2t = saW2.T.astype(bf16)
    saw3t = saW3.T.astype(bf16)

    wspec = lambda shp: pl.BlockSpec(shp, lambda bi, ti: (0,) * len(shp))
    _NB = 4

    def run_k1(xh, eh):
        return pl.pallas_call(
            _k1_body,
            grid=(_NB, _N // _TN),
            in_specs=[
                pl.BlockSpec((1, _TN, _D), lambda bi, ti: (bi, ti, 0)),
                pl.BlockSpec((1, _TN, 3), lambda bi, ti: (bi, ti, 0)),
                wspec((_D, _D)), wspec((1, _D)), wspec((1, _D)),
                wspec((1, _D)),
                wspec((_D, _D)), wspec((1, _D)), wspec((1, _D)),
                wspec((1, _D)),
                wspec((_D, 3)), wspec((1, 3)),
                wspec((_D, _D)), wspec((1, _D)),
                wspec((_D, _D)), wspec((1, _D)),
            ],
            out_specs=[
                pl.BlockSpec((1, _TN, 3), lambda bi, ti: (bi, ti, 0)),
                pl.BlockSpec((_TN, _D),
                             lambda bi, ti: (bi * (_N // _TN) + ti, 0)),
            ],
            out_shape=[
                jax.ShapeDtypeStruct((_NB, _N, 3), f32),
                jax.ShapeDtypeStruct((_NB * _N, _D), f32),
            ],
        )(xh, eh, w1t, b1[None, :], g1s, be1[None, :],
          w2t, b2[None, :], g2s, be2[None, :], w3a, b3a, w3d, b3d, at, sag1s)

    def run_k4(gf, gx):
        return pl.pallas_call(
            _k4_body,
            grid=(_NB,),
            in_specs=[
                pl.BlockSpec((1, _NS, _NQ, _D), lambda bi: (bi, 0, 0, 0)),
                pl.BlockSpec((1, _NS, _NQ, 3), lambda bi: (bi, 0, 0, 0)),
                pl.BlockSpec((3, _D), lambda bi: (0, 0)),
                pl.BlockSpec((1, _D), lambda bi: (0, 0)),
                pl.BlockSpec((1, _D), lambda bi: (0, 0)),
                pl.BlockSpec((_D, _D), lambda bi: (0, 0)),
                pl.BlockSpec((1, _D), lambda bi: (0, 0)),
                pl.BlockSpec((1, _D), lambda bi: (0, 0)),
                pl.BlockSpec((_D, _D), lambda bi: (0, 0)),
                pl.BlockSpec((1, _D), lambda bi: (0, 0)),
                pl.BlockSpec((1, _D), lambda bi: (0, 0)),
            ],
            out_specs=pl.BlockSpec((1, _NQ, _D), lambda bi: (bi, 0, 0)),
            out_shape=jax.ShapeDtypeStruct((_NB, _NQ, _D), f32),
        )(gf, gx, b3t, sag1s, sab1[None, :], saw2t, sag2s, sab2[None, :],
          saw3t, sag3s, sab3[None, :])

    exs = encode_xyz[..., 0]
    eys = encode_xyz[..., 1]
    ezs = encode_xyz[..., 2]
    sample_inds = pl.pallas_call(
        _fps_body,
        out_shape=jax.ShapeDtypeStruct((_B, _NQ), jnp.int32),
    )(exs, eys, ezs)

    ballq = _make_ballq(_NB)
    votes, newxs, qfs = [], [], []
    for h in range(_B // _NB):
        b0 = h * _NB
        vote_h, hrows_h = run_k1(xT[b0:b0 + _NB], encode_xyz[b0:b0 + _NB])
        new_xyz_h, gx_h, gf_h = ballq(
            vote_h[..., 0], vote_h[..., 1], vote_h[..., 2],
            sample_inds[b0:b0 + _NB], hrows_h)
        qfs.append(run_k4(gf_h, gx_h))
        votes.append(vote_h)
        newxs.append(new_xyz_h)

    vote_xyz = jnp.concatenate(votes, axis=0)
    new_xyz = jnp.concatenate(newxs, axis=0)
    query_features = jnp.transpose(jnp.concatenate(qfs, axis=0), (0, 2, 1))
    return (vote_xyz, encode_xyz, sample_inds, new_xyz, query_features)


# speedup vs baseline: 1.0041x; 1.0041x over previous
import functools

import numpy as np
import jax
import jax.numpy as jnp
from jax import lax
from jax.experimental import pallas as pl
from jax.experimental.pallas import tpu as pltpu
from jax.experimental.pallas import tpu_sc as plsc

_D = 288
_NQ = 256
_NS = 16
_B = 8
_N = 2048
_EPS = 1e-5
_R = 0.3
_TN = 1024


def _bf16_dot(a, w_ref):
    return jnp.dot(a.astype(jnp.bfloat16), w_ref[...],
                   preferred_element_type=jnp.float32)


def _k1_body(x_ref, ex_ref, w1_ref, b1_ref, g1_ref, be1_ref,
             w2_ref, b2_ref, g2_ref, be2_ref, w3a_ref, b3a_ref,
             w3d_ref, b3d_ref, at_ref, sag1_ref, vote_ref, hpre_ref):
    x = x_ref[0]
    h = (_bf16_dot(x, w1_ref) + b1_ref[...]) * g1_ref[...] + be1_ref[...]
    h = jnp.maximum(h, 0.0)
    h = (_bf16_dot(h, w2_ref) + b2_ref[...]) * g2_ref[...] + be2_ref[...]
    h = jnp.maximum(h, 0.0)
    out3 = _bf16_dot(h, w3a_ref) + b3a_ref[...]
    outd = _bf16_dot(h, w3d_ref) + b3d_ref[...]
    raw = x + outd
    ssq = jnp.sum(raw * raw, axis=1, keepdims=True)
    feats = raw / jnp.sqrt(ssq)
    hpre_ref[...] = _bf16_dot(feats, at_ref) * sag1_ref[...]
    shift = jnp.clip(jax.nn.sigmoid(out3), 0.1, 0.9)
    vote_ref[0] = ex_ref[0] + shift - 0.5


def _fps_body(xs_ref, ys_ref, zs_ref, out_ref):
    xs = xs_ref[...]
    ys = ys_ref[...]
    zs = zs_ref[...]
    iota = lax.broadcasted_iota(jnp.int32, (_B, _N), 1)
    iota_q = lax.broadcasted_iota(jnp.int32, (_B, _NQ), 1)

    def step(i, carry):
        dists, far = carry
        out_ref[...] = jnp.where(iota_q == i,
                                 jnp.broadcast_to(far, (_B, _NQ)),
                                 out_ref[...])
        oh = iota == far
        cx = jnp.sum(jnp.where(oh, xs, 0.0), axis=1, keepdims=True)
        cy = jnp.sum(jnp.where(oh, ys, 0.0), axis=1, keepdims=True)
        cz = jnp.sum(jnp.where(oh, zs, 0.0), axis=1, keepdims=True)
        d = (xs - cx) ** 2 + (ys - cy) ** 2 + (zs - cz) ** 2
        dists = jnp.minimum(dists, d)
        mx = jnp.max(dists, axis=1, keepdims=True)
        far = jnp.min(jnp.where(dists == mx, iota, _N), axis=1, keepdims=True)
        return dists, far

    lax.fori_loop(0, _NQ, step,
                  (jnp.full((_B, _N), 1e10, jnp.float32),
                   jnp.zeros((_B, 1), jnp.int32)))


def _ballq_body(nb, vx_hbm, vy_hbm, vz_hbm, sinds_hbm, hrows_hbm,
                nxyz_hbm, gx_hbm, gf_hbm,
                px, py, pz, sq, qx, qy, qz, nbuf, lst, gxbuf, glist,
                rb0, rb1, rb2, rb3, gsems, osems):
    wpb = 32 // nb
    _QW = _NQ // wpb
    wid = lax.axis_index("s") * 2 + lax.axis_index("c")
    b = wid // wpb
    s0 = (wid % wpb) * _QW
    pltpu.sync_copy(vx_hbm.at[b], px)
    pltpu.sync_copy(vy_hbm.at[b], py)
    pltpu.sync_copy(vz_hbm.at[b], pz)
    pltpu.sync_copy(sinds_hbm.at[b, pl.ds(s0, _QW)], sq)

    lanes = lax.broadcasted_iota(jnp.int32, (16,), 0)
    zeros16 = jnp.zeros((16,), jnp.int32)
    ones16 = jnp.full((16,), 1, jnp.int32)
    twos16 = jnp.full((16,), 2, jnp.int32)

    for g in range(_QW // 16):
        qi = sq[pl.ds(g * 16, 16)]
        gqx = plsc.load_gather(px, [qi])
        gqy = plsc.load_gather(py, [qi])
        gqz = plsc.load_gather(pz, [qi])
        qx[pl.ds(g * 16, 16)] = gqx
        qy[pl.ds(g * 16, 16)] = gqy
        qz[pl.ds(g * 16, 16)] = gqz
        rid = lanes + g * 16
        plsc.store_scatter(nbuf, [rid, zeros16], gqx)
        plsc.store_scatter(nbuf, [rid, ones16], gqy)
        plsc.store_scatter(nbuf, [rid, twos16], gqz)
    pltpu.sync_copy(nbuf, nxyz_hbm.at[b, pl.ds(s0, _QW), :])

    r2 = jnp.float32(_R * _R)
    inv_r = jnp.float32(_R)
    _UNROLL = 4

    def per_query(qq, _):
        qxb = jnp.full((16,), qx[pl.ds(qq, 16)][0], jnp.float32)
        qyb = jnp.full((16,), qy[pl.ds(qq, 16)][0], jnp.float32)
        qzb = jnp.full((16,), qz[pl.ds(qq, 16)][0], jnp.float32)

        def chunk(cu, cursor):
            for u in range(_UNROLL):
                off = cu * (16 * _UNROLL) + u * 16
                pxv = px[pl.ds(off, 16)]
                pyv = py[pl.ds(off, 16)]
                pzv = pz[pl.ds(off, 16)]
                dx = pxv - qxb
                dy = pyv - qyb
                dz = pzv - qzb
                d2 = dx * dx + dy * dy + dz * dz
                m = d2 < r2
                cnt = plsc.all_reduce_population_count(m)[0]
                slot = jnp.minimum(cursor, 16)
                plsc.store_compressed(lst.at[pl.ds(slot, 16)],
                                      lanes + off, mask=m)
                cursor = cursor + cnt
            return cursor

        cursor = lax.fori_loop(0, _N // (16 * _UNROLL), chunk, jnp.int32(0))
        cnt16 = jnp.minimum(cursor, 16)
        nbv = lst[pl.ds(0, 16)]
        first = jnp.full((16,), nbv[0], jnp.int32)
        nb = jnp.where(lanes < cnt16, nbv, first)
        gxx = (plsc.load_gather(px, [nb]) - qxb) / inv_r
        gxy = (plsc.load_gather(py, [nb]) - qyb) / inv_r
        gxz = (plsc.load_gather(pz, [nb]) - qzb) / inv_r
        qqv = jnp.full((16,), qq, jnp.int32)
        plsc.store_scatter(gxbuf, [lanes, qqv, zeros16], gxx)
        plsc.store_scatter(gxbuf, [lanes, qqv, ones16], gxy)
        plsc.store_scatter(gxbuf, [lanes, qqv, twos16], gxz)
        plsc.store_scatter(glist, [lanes * _QW + qqv], nb + b * _N)
        return 0

    lax.fori_loop(0, _QW, per_query, 0)
    pltpu.sync_copy(gxbuf, gx_hbm.at[b, :, pl.ds(s0, _QW), :])

    bufs = [rb0, rb1, rb2, rb3]

    def _gather(k):
        return pltpu.async_copy(
            hrows_hbm.at[glist.at[pl.ds(k * _QW, _QW)]],
            bufs[k % 4], gsems.at[k % 4])

    def _out(k):
        return pltpu.async_copy(
            bufs[k % 4], gf_hbm.at[b, k, pl.ds(s0, _QW), :],
            osems.at[k % 4])

    gcp = [None] * _NS
    ocp = [None] * _NS
    for k in range(4):
        gcp[k] = _gather(k)
    for k in range(_NS):
        gcp[k].wait()
        ocp[k] = _out(k)
        nk = k + 2
        if 4 <= nk < _NS:
            ocp[nk - 4].wait()
            gcp[nk] = _gather(nk)
    for k in range(_NS - 4, _NS):
        ocp[k].wait()


def _k4_body(gf_ref, gx_ref, b3t_ref, sag1_ref, sab1_ref, w2_ref, sag2_ref,
             sab2_ref, w3_ref, sag3_ref, sab3_ref, out_ref):
    acc = jnp.zeros((_NQ, _D), jnp.float32)
    b3t = b3t_ref[...]
    for k in range(_NS):
        g = gf_ref[0, k]
        gxk = gx_ref[0, k].astype(jnp.bfloat16).astype(jnp.float32)
        xyzp = (gxk[:, 0:1] * b3t[0:1, :] + gxk[:, 1:2] * b3t[1:2, :]
                + gxk[:, 2:3] * b3t[2:3, :]) * sag1_ref[...]
        l1 = jnp.maximum(g + xyzp + sab1_ref[...], 0.0)
        l2 = jnp.maximum(
            _bf16_dot(l1, w2_ref) * sag2_ref[...] + sab2_ref[...], 0.0)
        l3 = jnp.maximum(
            _bf16_dot(l2, w3_ref) * sag3_ref[...] + sab3_ref[...], 0.0)
        acc = jnp.maximum(acc, l3)
    out_ref[0] = acc


def _make_ballq(nb):
    qw = _NQ // (32 // nb)
    mesh = plsc.VectorSubcoreMesh(core_axis_name="c", subcore_axis_name="s")
    return functools.partial(
        pl.kernel,
        out_type=(
            jax.ShapeDtypeStruct((nb, _NQ, 3), jnp.float32),
            jax.ShapeDtypeStruct((nb, _NS, _NQ, 3), jnp.float32),
            jax.ShapeDtypeStruct((nb, _NS, _NQ, _D), jnp.float32),
        ),
        mesh=mesh,
        compiler_params=pltpu.CompilerParams(needs_layout_passes=False,
                                             use_tc_tiling_on_sc=False),
        scratch_types=[
            pltpu.VMEM((_N,), jnp.float32),
            pltpu.VMEM((_N,), jnp.float32),
            pltpu.VMEM((_N,), jnp.float32),
            pltpu.VMEM((qw,), jnp.int32),
            pltpu.VMEM((qw + 16,), jnp.float32),
            pltpu.VMEM((qw + 16,), jnp.float32),
            pltpu.VMEM((qw + 16,), jnp.float32),
            pltpu.VMEM((qw, 3), jnp.float32),
            pltpu.VMEM((32,), jnp.int32),
            pltpu.VMEM((_NS, qw, 3), jnp.float32),
            pltpu.VMEM((_NS * qw,), jnp.int32),
            pltpu.VMEM((qw, _D), jnp.float32),
            pltpu.VMEM((qw, _D), jnp.float32),
            pltpu.VMEM((qw, _D), jnp.float32),
            pltpu.VMEM((qw, _D), jnp.float32),
            pltpu.SemaphoreType.DMA((4,)),
            pltpu.SemaphoreType.DMA((4,)),
        ],
    )(functools.partial(_ballq_body, nb))


def kernel(encode_xyz, encode_features, W1, b1, g1, be1, W2, b2, g2, be2,
           W3, b3, saW1, sag1, sab1, saW2, sag2, sab2, saW3, sag3, sab3):
    f32 = jnp.float32
    bf16 = jnp.bfloat16
    scale = f32(1.0 / np.sqrt(1.0 + _EPS))
    g1s = (g1 * scale)[None, :]
    g2s = (g2 * scale)[None, :]
    sag1s = (sag1 * scale)[None, :]
    sag2s = (sag2 * scale)[None, :]
    sag3s = (sag3 * scale)[None, :]

    xT = jnp.transpose(encode_features, (0, 2, 1))
    w1t = W1.T.astype(bf16)
    w2t = W2.T.astype(bf16)
    w3a = W3[:3].T.astype(bf16)
    b3a = b3[:3][None, :]
    w3d = W3[3:].T.astype(bf16)
    b3d = b3[3:][None, :]
    at = saW1[:, 3:].T.astype(bf16)
    b3t = saW1[:, :3].T.astype(bf16).astype(f32)
    saw2t = saW2.T.astype(bf16)
    saw3t = saW3.T.astype(bf16)

    wspec = lambda shp: pl.BlockSpec(shp, lambda bi, ti: (0,) * len(shp))
    _NB = 4

    def run_k1(xh, eh):
        return pl.pallas_call(
            _k1_body,
            grid=(_NB, _N // _TN),
            in_specs=[
                pl.BlockSpec((1, _TN, _D), lambda bi, ti: (bi, ti, 0)),
                pl.BlockSpec((1, _TN, 3), lambda bi, ti: (bi, ti, 0)),
                wspec((_D, _D)), wspec((1, _D)), wspec((1, _D)),
                wspec((1, _D)),
                wspec((_D, _D)), wspec((1, _D)), wspec((1, _D)),
                wspec((1, _D)),
                wspec((_D, 3)), wspec((1, 3)),
                wspec((_D, _D)), wspec((1, _D)),
                wspec((_D, _D)), wspec((1, _D)),
            ],
            out_specs=[
                pl.BlockSpec((1, _TN, 3), lambda bi, ti: (bi, ti, 0)),
                pl.BlockSpec((_TN, _D),
                             lambda bi, ti: (bi * (_N // _TN) + ti, 0)),
            ],
            out_shape=[
                jax.ShapeDtypeStruct((_NB, _N, 3), f32),
                jax.ShapeDtypeStruct((_NB * _N, _D), f32),
            ],
        )(xh, eh, w1t, b1[None, :], g1s, be1[None, :],
          w2t, b2[None, :], g2s, be2[None, :], w3a, b3a, w3d, b3d, at, sag1s)

    def run_k4(gf, gx):
        return pl.pallas_call(
            _k4_body,
            grid=(_NB,),
            in_specs=[
                pl.BlockSpec((1, _NS, _NQ, _D), lambda bi: (bi, 0, 0, 0)),
                pl.BlockSpec((1, _NS, _NQ, 3), lambda bi: (bi, 0, 0, 0)),
                pl.BlockSpec((3, _D), lambda bi: (0, 0)),
                pl.BlockSpec((1, _D), lambda bi: (0, 0)),
                pl.BlockSpec((1, _D), lambda bi: (0, 0)),
                pl.BlockSpec((_D, _D), lambda bi: (0, 0)),
                pl.BlockSpec((1, _D), lambda bi: (0, 0)),
                pl.BlockSpec((1, _D), lambda bi: (0, 0)),
                pl.BlockSpec((_D, _D), lambda bi: (0, 0)),
                pl.BlockSpec((1, _D), lambda bi: (0, 0)),
                pl.BlockSpec((1, _D), lambda bi: (0, 0)),
            ],
            out_specs=pl.BlockSpec((1, _NQ, _D), lambda bi: (bi, 0, 0)),
            out_shape=jax.ShapeDtypeStruct((_NB, _NQ, _D), f32),
        )(gf, gx, b3t, sag1s, sab1[None, :], saw2t, sag2s, sab2[None, :],
          saw3t, sag3s, sab3[None, :])

    exs = encode_xyz[..., 0]
    eys = encode_xyz[..., 1]
    ezs = encode_xyz[..., 2]
    sample_inds = pl.pallas_call(
        _fps_body,
        out_shape=jax.ShapeDtypeStruct((_B, _NQ), jnp.int32),
    )(exs, eys, ezs)

    ballq = _make_ballq(_NB)
    votes, newxs, qfs = [], [], []
    for h in range(_B // _NB):
        b0 = h * _NB
        vote_h, hrows_h = run_k1(xT[b0:b0 + _NB], encode_xyz[b0:b0 + _NB])
        new_xyz_h, gx_h, gf_h = ballq(
            vote_h[..., 0], vote_h[..., 1], vote_h[..., 2],
            sample_inds[b0:b0 + _NB], hrows_h)
        qfs.append(run_k4(gf_h, gx_h))
        votes.append(vote_h)
        newxs.append(new_xyz_h)

    vote_xyz = jnp.concatenate(votes, axis=0)
    new_xyz = jnp.concatenate(newxs, axis=0)
    query_features = jnp.transpose(jnp.concatenate(qfs, axis=0), (0, 2, 1))
    return (vote_xyz, encode_xyz, sample_inds, new_xyz, query_features)
